# baseline (speedup 1.0000x reference)
import functools

import jax
import jax.numpy as jnp
from jax import lax
from jax.experimental import pallas as pl
from jax.experimental.pallas import tpu as pltpu
from jax.experimental.pallas import tpu_sc as plsc

_B = 16384
_F = 64
_GLOBAL_MEAN = 3.5

_NC = 2
_NS = 16
_NW = _NC * _NS
_BPW = _B // _NW
_CH = 128
_NCHUNK = _BPW // _CH
_GRP = _BPW // 16


def _body(uid_hbm, iid_hbm, uf_hbm, if_hbm, ub_hbm, ib_hbm, out_hbm,
          uidx_v, iidx_v, urows_v, irows_v, ubias_v, ibias_v, out_v,
          *sems):
    wid = lax.axis_index("s") * _NC + lax.axis_index("c")
    base = wid * _BPW

    for j in range(_NCHUNK):
        pltpu.sync_copy(uid_hbm.at[pl.ds(base + j * _CH, _CH)], uidx_v.at[j])
        pltpu.sync_copy(iid_hbm.at[pl.ds(base + j * _CH, _CH)], iidx_v.at[j])

    copies = []
    for j in range(_NCHUNK):
        sem = sems[j]
        sl = pl.ds(j * _CH, _CH)
        copies.append((
            pltpu.async_copy(uf_hbm.at[uidx_v.at[j]], urows_v.at[sl], sem),
            pltpu.async_copy(if_hbm.at[iidx_v.at[j]], irows_v.at[sl], sem),
            pltpu.async_copy(ub_hbm.at[uidx_v.at[j]], ubias_v.at[sl], sem),
            pltpu.async_copy(ib_hbm.at[iidx_v.at[j]], ibias_v.at[sl], sem),
        ))

    for j in range(_NCHUNK):
        for c in copies[j]:
            c.wait()

        def group_body(g, _, j=j):
            r0 = j * _CH + g * 16
            rows = lax.iota(jnp.int32, 16) + r0
            acc = ubias_v[pl.ds(r0, 16)] + ibias_v[pl.ds(r0, 16)] + _GLOBAL_MEAN

            def f_body(fb, acc):
                for ff in range(8):
                    col = jnp.full((16,), fb * 8 + ff, jnp.int32)
                    u = plsc.load_gather(urows_v, [rows, col])
                    q = plsc.load_gather(irows_v, [rows, col])
                    acc = acc + u * q
                return acc

            acc = lax.fori_loop(0, _F // 8, f_body, acc)
            out_v[pl.ds(r0, 16)] = acc
            return 0

        lax.fori_loop(0, _CH // 16, group_body, 0)

    pltpu.sync_copy(out_v, out_hbm.at[pl.ds(base, _BPW)])


@jax.jit
def _svd_predict(user_ids, item_ids, user_factors, item_factors,
                 user_bias, item_bias):
    mesh = plsc.VectorSubcoreMesh(core_axis_name="c", subcore_axis_name="s")
    run = pl.kernel(
        _body,
        out_type=jax.ShapeDtypeStruct((_B,), jnp.float32),
        mesh=mesh,
        scratch_types=[
            pltpu.VMEM((_NCHUNK, _CH), jnp.int32),
            pltpu.VMEM((_NCHUNK, _CH), jnp.int32),
            pltpu.VMEM((_BPW, _F), jnp.float32),
            pltpu.VMEM((_BPW, _F), jnp.float32),
            pltpu.VMEM((_BPW,), jnp.float32),
            pltpu.VMEM((_BPW,), jnp.float32),
            pltpu.VMEM((_BPW,), jnp.float32),
        ] + [pltpu.SemaphoreType.DMA] * _NCHUNK,
        compiler_params=pltpu.CompilerParams(needs_layout_passes=False,
                                             use_tc_tiling_on_sc=False),
    )
    return run(user_ids, item_ids, user_factors, item_factors,
               user_bias, item_bias)


def kernel(user_ids, item_ids, user_factors, item_factors, user_bias,
           item_bias):
    return _svd_predict(
        user_ids.astype(jnp.int32),
        item_ids.astype(jnp.int32),
        user_factors,
        item_factors,
        user_bias.reshape(-1),
        item_bias.reshape(-1),
    )
